# baseline (speedup 1.0000x reference)
import functools

import jax
import jax.numpy as jnp
from jax import lax
from jax.experimental import pallas as pl
from jax.experimental.pallas import tpu as pltpu
from jax.experimental.pallas import tpu_sc as plsc

N_FIELDS = 26
VOCAB = 100000
EMBED = 32
N_NUM = 13
N_TOK = N_FIELDS + N_NUM
LANES = 16


@functools.partial(jax.jit, static_argnames=("batch",))
def _run(xcat_t, xnum_t, tbl, w, bvec, *, batch):
    info = plsc.get_sparse_core_info()
    nc, ns = info.num_cores, info.num_subcores
    nw = nc * ns
    rows_per_w = batch // nw
    chunk = 64
    n_chunks = rows_per_w // chunk

    mesh = plsc.VectorSubcoreMesh(core_axis_name="c", subcore_axis_name="s")

    def body(xcat_ref, xnum_ref, tbl_ref, w_ref, b_ref, out_ref,
             idx_v, x_v, g_v, n_v, w_v, b_v):
        wid = lax.axis_index("s") * nc + lax.axis_index("c")

        pltpu.sync_copy(w_ref, w_v)
        pltpu.sync_copy(b_ref, b_v)
        wlo = w_v[pl.ds(0, LANES)]
        whi = w_v[pl.ds(LANES, LANES)]
        blo = b_v[pl.ds(0, LANES)]
        bhi = b_v[pl.ds(LANES, LANES)]

        def chunk_body(ci, carry):
            base = wid * rows_per_w + ci * chunk

            pltpu.sync_copy(xcat_ref.at[:, pl.ds(base, chunk)], idx_v)
            pltpu.sync_copy(xnum_ref.at[:, pl.ds(base, chunk)], x_v)

            def f_off(f, c):
                off = f * VOCAB

                def k_body(k, cc):
                    sl = pl.ds(k * LANES, LANES)
                    idx_v[f, sl] = idx_v[f, sl] + off
                    return cc

                return lax.fori_loop(0, chunk // LANES, k_body, c)

            lax.fori_loop(0, N_FIELDS, f_off, carry)

            def f_gather(f, c):
                pltpu.sync_copy(tbl_ref.at[idx_v.at[f]], g_v.at[f])
                return c

            lax.fori_loop(0, N_FIELDS, f_gather, carry)

            def f_out(f, c):
                pltpu.sync_copy(g_v.at[f], out_ref.at[pl.ds(base, chunk), f])
                return c

            lax.fori_loop(0, N_FIELDS, f_out, carry)

            def i_body(i, c):
                def r_body(r, cc):
                    ii = jnp.full((LANES,), i, jnp.int32)
                    rr = jnp.full((LANES,), r, jnp.int32)
                    s = plsc.load_gather(x_v, [ii, rr])
                    n_v[i, r, pl.ds(0, LANES)] = s * wlo + blo
                    n_v[i, r, pl.ds(LANES, LANES)] = s * whi + bhi
                    return cc

                return lax.fori_loop(0, chunk, r_body, c)

            lax.fori_loop(0, N_NUM, i_body, carry)

            def i_out(i, c):
                pltpu.sync_copy(n_v.at[i],
                                out_ref.at[pl.ds(base, chunk), N_FIELDS + i])
                return c

            lax.fori_loop(0, N_NUM, i_out, carry)
            return carry

        lax.fori_loop(0, n_chunks, chunk_body, 0)

    call = pl.kernel(
        body,
        out_type=jax.ShapeDtypeStruct((batch, N_TOK, EMBED), jnp.float32),
        mesh=mesh,
        scratch_types=[
            pltpu.VMEM((N_FIELDS, chunk), jnp.int32),
            pltpu.VMEM((N_NUM, chunk), jnp.float32),
            pltpu.VMEM((N_FIELDS, chunk, EMBED), jnp.float32),
            pltpu.VMEM((N_NUM, chunk, EMBED), jnp.float32),
            pltpu.VMEM((EMBED,), jnp.float32),
            pltpu.VMEM((EMBED,), jnp.float32),
        ],
        compiler_params=pltpu.CompilerParams(
            use_tc_tiling_on_sc=False, needs_layout_passes=False),
    )
    return call(xcat_t, xnum_t, tbl, w, bvec)


def kernel(x_cat, x_num, tables, W, b):
    batch = x_cat.shape[0]
    xcat_t = x_cat.astype(jnp.int32).T
    xnum_t = x_num.T
    tbl = tables.reshape(N_FIELDS * VOCAB, EMBED)
    w = W.reshape(EMBED)
    return _run(xcat_t, xnum_t, tbl, w, b, batch=batch)


# speedup vs baseline: 1.1044x; 1.1044x over previous
import functools

import jax
import jax.numpy as jnp
from jax import lax
from jax.experimental import pallas as pl
from jax.experimental.pallas import tpu as pltpu
from jax.experimental.pallas import tpu_sc as plsc

N_FIELDS = 26
VOCAB = 100000
EMBED = 32
N_NUM = 13
N_TOK = N_FIELDS + N_NUM
LANES = 16


@functools.partial(jax.jit, static_argnames=("batch",))
def _run(xcat_t, xnum_t, tbl, w, bvec, *, batch):
    info = plsc.get_sparse_core_info()
    nc, ns = info.num_cores, info.num_subcores
    nw = nc * ns
    rows_per_w = batch // nw
    chunk = 64
    n_chunks = rows_per_w // chunk

    mesh = plsc.VectorSubcoreMesh(core_axis_name="c", subcore_axis_name="s")

    def body(xcat_ref, xnum_ref, tbl_ref, w_ref, b_ref, out_ref,
             idx_v, x_v, g_v, n_v, w_v, b_v, gsem, wsem):
        wid = lax.axis_index("s") * nc + lax.axis_index("c")

        pltpu.sync_copy(w_ref, w_v)
        pltpu.sync_copy(b_ref, b_v)
        wlo = w_v[pl.ds(0, LANES)]
        whi = w_v[pl.ds(LANES, LANES)]
        blo = b_v[pl.ds(0, LANES)]
        bhi = b_v[pl.ds(LANES, LANES)]

        def drain_writes(base, c):
            def f_wd(f, cc):
                pltpu.make_async_copy(
                    g_v.at[f], out_ref.at[pl.ds(base, chunk), f], wsem).wait()
                return cc

            lax.fori_loop(0, N_FIELDS, f_wd, c)

            def i_wd(i, cc):
                pltpu.make_async_copy(
                    n_v.at[i], out_ref.at[pl.ds(base, chunk), N_FIELDS + i],
                    wsem).wait()
                return cc

            lax.fori_loop(0, N_NUM, i_wd, c)

        def chunk_body(ci, carry):
            base = wid * rows_per_w + ci * chunk

            pltpu.sync_copy(xcat_ref.at[:, pl.ds(base, chunk)], idx_v)
            pltpu.sync_copy(xnum_ref.at[:, pl.ds(base, chunk)], x_v)

            def f_off(f, c):
                off = f * VOCAB

                def k_body(k, cc):
                    sl = pl.ds(k * LANES, LANES)
                    idx_v[f, sl] = idx_v[f, sl] + off
                    return cc

                return lax.fori_loop(0, chunk // LANES, k_body, c)

            lax.fori_loop(0, N_FIELDS, f_off, carry)

            @pl.when(ci > 0)
            def _():
                drain_writes(base - chunk, carry)

            def f_fire(f, c):
                pltpu.async_copy(tbl_ref.at[idx_v.at[f]], g_v.at[f], gsem)
                return c

            lax.fori_loop(0, N_FIELDS, f_fire, carry)

            def i_body(i, c):
                ii = jnp.full((LANES,), i, jnp.int32)

                def r_body(r, cc):
                    rr = jnp.full((LANES,), r, jnp.int32)
                    s = plsc.load_gather(x_v, [ii, rr])
                    n_v[i, r, pl.ds(0, LANES)] = s * wlo + blo
                    n_v[i, r, pl.ds(LANES, LANES)] = s * whi + bhi
                    return cc

                return lax.fori_loop(0, chunk, r_body, c)

            lax.fori_loop(0, N_NUM, i_body, carry)

            def i_out(i, c):
                pltpu.async_copy(n_v.at[i],
                                 out_ref.at[pl.ds(base, chunk), N_FIELDS + i],
                                 wsem)
                return c

            lax.fori_loop(0, N_NUM, i_out, carry)

            def f_drain(f, c):
                pltpu.make_async_copy(
                    tbl_ref.at[idx_v.at[f]], g_v.at[f], gsem).wait()
                return c

            lax.fori_loop(0, N_FIELDS, f_drain, carry)

            def f_out(f, c):
                pltpu.async_copy(g_v.at[f],
                                 out_ref.at[pl.ds(base, chunk), f], wsem)
                return c

            lax.fori_loop(0, N_FIELDS, f_out, carry)
            return carry

        lax.fori_loop(0, n_chunks, chunk_body, 0)
        drain_writes((wid + 1) * rows_per_w - chunk, 0)

    call = pl.kernel(
        body,
        out_type=jax.ShapeDtypeStruct((batch, N_TOK, EMBED), jnp.float32),
        mesh=mesh,
        scratch_types=[
            pltpu.VMEM((N_FIELDS, chunk), jnp.int32),
            pltpu.VMEM((N_NUM, chunk), jnp.float32),
            pltpu.VMEM((N_FIELDS, chunk, EMBED), jnp.float32),
            pltpu.VMEM((N_NUM, chunk, EMBED), jnp.float32),
            pltpu.VMEM((EMBED,), jnp.float32),
            pltpu.VMEM((EMBED,), jnp.float32),
            pltpu.SemaphoreType.DMA,
            pltpu.SemaphoreType.DMA,
        ],
        compiler_params=pltpu.CompilerParams(
            use_tc_tiling_on_sc=False, needs_layout_passes=False),
    )
    return call(xcat_t, xnum_t, tbl, w, bvec)


def kernel(x_cat, x_num, tables, W, b):
    batch = x_cat.shape[0]
    xcat_t = x_cat.astype(jnp.int32).T
    xnum_t = x_num.T
    tbl = tables.reshape(N_FIELDS * VOCAB, EMBED)
    w = W.reshape(EMBED)
    return _run(xcat_t, xnum_t, tbl, w, b, batch=batch)
